# Initial kernel scaffold; baseline (speedup 1.0000x reference)
import jax
import jax.numpy as jnp
from jax.experimental import pallas as pl


def kernel(inputs, embeddings):
    raise NotImplementedError("write your pallas kernel here")


# speedup vs baseline: 1.4785x; 1.4785x over previous
import functools

import jax
import jax.numpy as jnp
from jax import lax
from jax.experimental import pallas as pl
from jax.experimental.pallas import tpu as pltpu
from jax.experimental.pallas import tpu_sc as plsc

BATCH = 4096
LENGTH = 200
DIM = 32
B = BATCH * LENGTH
NC, NS = 2, 16
NW = NC * NS
BPW = B // NW
CHUNK = 1600
NCHUNK = BPW // CHUNK

_mesh = plsc.VectorSubcoreMesh(
    core_axis_name="c", subcore_axis_name="s", num_cores=NC, num_subcores=NS
)


@functools.partial(
    pl.kernel,
    out_type=jax.ShapeDtypeStruct((B, DIM), jnp.float32),
    mesh=_mesh,
    scratch_types=[
        pltpu.VMEM((CHUNK,), jnp.int32),
        pltpu.VMEM((CHUNK, DIM), jnp.float32),
        pltpu.SemaphoreType.DMA,
    ],
    compiler_params=pltpu.CompilerParams(use_tc_tiling_on_sc=False),
)
def _gather_kernel(ids_hbm, table_hbm, out_hbm, idx_v, rows_v, sem):
    wid = lax.axis_index("s") * NC + lax.axis_index("c")
    base = wid * BPW

    def body(i, carry):
        off = base + i * CHUNK
        pltpu.sync_copy(ids_hbm.at[pl.ds(off, CHUNK)], idx_v)
        pltpu.async_copy(table_hbm.at[idx_v], rows_v, sem).wait()
        pltpu.sync_copy(rows_v, out_hbm.at[pl.ds(off, CHUNK)])
        return carry

    lax.fori_loop(0, NCHUNK, body, 0)


def kernel(inputs, embeddings):
    ids = jnp.reshape(inputs, (B,)).astype(jnp.int32)
    out = _gather_kernel(ids, embeddings)
    return jnp.reshape(out, (BATCH, LENGTH, DIM))


# speedup vs baseline: 1.5066x; 1.0190x over previous
import functools

import jax
import jax.numpy as jnp
from jax import lax
from jax.experimental import pallas as pl
from jax.experimental.pallas import tpu as pltpu
from jax.experimental.pallas import tpu_sc as plsc

BATCH = 4096
LENGTH = 200
DIM = 32
B = BATCH * LENGTH
NC, NS = 2, 16
NW = NC * NS
BPW = B // NW
CHUNK = 1600
NCHUNK = BPW // CHUNK

_mesh = plsc.VectorSubcoreMesh(
    core_axis_name="c", subcore_axis_name="s", num_cores=NC, num_subcores=NS
)


@functools.partial(
    pl.kernel,
    out_type=jax.ShapeDtypeStruct((B, DIM), jnp.float32),
    mesh=_mesh,
    scratch_types=[
        pltpu.VMEM((CHUNK,), jnp.int32),
        pltpu.VMEM((CHUNK,), jnp.int32),
        pltpu.VMEM((CHUNK, DIM), jnp.float32),
        pltpu.VMEM((CHUNK, DIM), jnp.float32),
        pltpu.SemaphoreType.DMA,
        pltpu.SemaphoreType.DMA,
        pltpu.SemaphoreType.DMA,
        pltpu.SemaphoreType.DMA,
        pltpu.SemaphoreType.DMA,
        pltpu.SemaphoreType.DMA,
    ],
    compiler_params=pltpu.CompilerParams(use_tc_tiling_on_sc=False),
)
def _gather_kernel(ids_hbm, table_hbm, out_hbm, idx_a, idx_b, rows_a,
                   rows_b, si_a, si_b, sg_a, sg_b, so_a, so_b):
    idx = (idx_a, idx_b)
    rows = (rows_a, rows_b)
    si = (si_a, si_b)
    sg = (sg_a, sg_b)
    so = (so_a, so_b)

    wid = lax.axis_index("s") * NC + lax.axis_index("c")
    base = wid * BPW

    def ids_slice(j):
        return ids_hbm.at[pl.ds(base + j * CHUNK, CHUNK)]

    def out_slice(j):
        return out_hbm.at[pl.ds(base + j * CHUNK, CHUNK)]

    pltpu.async_copy(ids_slice(0), idx[0], si[0])

    for j in range(NCHUNK):
        b = j % 2
        o = (j + 1) % 2
        pltpu.make_async_copy(ids_slice(j), idx[b], si[b]).wait()
        if j >= 2:
            pltpu.make_async_copy(rows[b], out_slice(j - 2), so[b]).wait()
        pltpu.async_copy(table_hbm.at[idx[b]], rows[b], sg[b])
        if j >= 1:
            pltpu.make_async_copy(table_hbm.at[idx[o]], rows[o], sg[o]).wait()
            pltpu.async_copy(rows[o], out_slice(j - 1), so[o])
        if j + 1 < NCHUNK:
            pltpu.async_copy(ids_slice(j + 1), idx[o], si[o])

    last = NCHUNK - 1
    lb = last % 2
    lo = (last + 1) % 2
    pltpu.make_async_copy(table_hbm.at[idx[lb]], rows[lb], sg[lb]).wait()
    pltpu.async_copy(rows[lb], out_slice(last), so[lb])
    pltpu.make_async_copy(rows[lo], out_slice(last - 1), so[lo]).wait()
    pltpu.make_async_copy(rows[lb], out_slice(last), so[lb]).wait()


def kernel(inputs, embeddings):
    ids = jnp.reshape(inputs, (B,)).astype(jnp.int32)
    out = _gather_kernel(ids, embeddings)
    return jnp.reshape(out, (BATCH, LENGTH, DIM))
